# baseline (speedup 1.0000x reference)
import functools

import jax
import jax.numpy as jnp
from jax import lax
from jax.experimental import pallas as pl
from jax.experimental.pallas import tpu as pltpu
from jax.experimental.pallas import tpu_sc as plsc

N = 10000
NPAD = 10240
NW8 = NPAD // 8
E = 160000
NW = 32
EPT = E // NW
CH = 128
K = EPT // CH
TAIL = EPT - K * CH
EPT_D = E // 16
K_D = EPT_D // CH
TAIL_D = EPT_D - K_D * CH
RPT = NPAD // 16
RPW = NPAD // 32

_mesh = plsc.VectorSubcoreMesh(core_axis_name="c", subcore_axis_name="s")
_sc_params = pltpu.CompilerParams(use_tc_tiling_on_sc=False,
                                  needs_layout_passes=False)


@functools.partial(
    pl.kernel,
    out_type=jax.ShapeDtypeStruct((NPAD, 16), jnp.float32),
    mesh=_mesh,
    compiler_params=_sc_params,
    scratch_types=[
        pltpu.VMEM((EPT_D,), jnp.int32),
        pltpu.VMEM((128,), jnp.float32),
        pltpu.VMEM((RPT,), jnp.float32),
        pltpu.VMEM((RPW,), jnp.float32),
        pltpu.VMEM((RPW, 16), jnp.float32),
        pltpu.VMEM_SHARED((NPAD,), jnp.float32),
        pltpu.SemaphoreType.DMA,
    ],
)
def _deg_kernel(ei_hbm, dinvw_hbm,
                idx_v, ones_v, zer_v, dinv_v, dvw_v, deg_sh, sem):
    c = lax.axis_index("c")
    s = lax.axis_index("s")
    wid = c * 16 + s
    one = jnp.ones((16,), jnp.float32)
    zero = jnp.zeros((16,), jnp.float32)

    def fill_ones(i, _):
        ones_v[pl.ds(i * 16, 16)] = one
        return 0

    lax.fori_loop(0, 8, fill_ones, 0)

    def fill_zero(i, _):
        zer_v[pl.ds(i * 16, 16)] = zero
        return 0

    lax.fori_loop(0, RPT // 16, fill_zero, 0)
    pltpu.sync_copy(ei_hbm.at[pl.ds(E + s * EPT_D, EPT_D)], idx_v)
    pltpu.sync_copy(zer_v, deg_sh.at[pl.ds(s * RPT, RPT)])
    plsc.subcore_barrier()

    def fire(j, _):
        pltpu.async_copy(ones_v.at[pl.ds(0, CH)],
                         deg_sh.at[idx_v.at[pl.ds(j * CH, CH)]], sem, add=True)
        return 0

    lax.fori_loop(0, K_D, fire, 0)
    pltpu.async_copy(ones_v.at[pl.ds(0, TAIL_D)],
                     deg_sh.at[idx_v.at[pl.ds(K_D * CH, TAIL_D)]], sem,
                     add=True)

    def drain(j, _):
        pltpu.make_async_copy(ones_v.at[pl.ds(0, CH)],
                              deg_sh.at[idx_v.at[pl.ds(j * CH, CH)]],
                              sem).wait()
        return 0

    lax.fori_loop(0, K_D, drain, 0)
    pltpu.make_async_copy(ones_v.at[pl.ds(0, TAIL_D)],
                          deg_sh.at[idx_v.at[pl.ds(K_D * CH, TAIL_D)]],
                          sem).wait()
    plsc.subcore_barrier()

    pltpu.sync_copy(deg_sh.at[pl.ds(wid * RPW, RPW)], dinv_v)

    def rsqrt_chunk(i, _):
        d = dinv_v[pl.ds(i * 16, 16)] + 1.0
        bits = plsc.bitcast(d, jnp.int32)
        y = plsc.bitcast(0x5F3759DF - lax.shift_right_logical(bits, 1),
                         jnp.float32)
        half = -0.5 * d
        y = y * (1.5 + half * y * y)
        y = y * (1.5 + half * y * y)
        y = y * (1.5 + half * y * y)
        dinv_v[pl.ds(i * 16, 16)] = y
        return 0

    lax.fori_loop(0, RPW // 16, rsqrt_chunk, 0)

    def splat_row(r, _):
        dvw_v[r] = plsc.load_gather(dinv_v, [jnp.full((16,), r, jnp.int32)])
        return 0

    lax.fori_loop(0, RPW, splat_row, 0)
    pltpu.sync_copy(dvw_v, dinvw_hbm.at[pl.ds(wid * RPW, RPW)])


@functools.partial(
    pl.kernel,
    out_type=jax.ShapeDtypeStruct((2, NPAD, 16), jnp.float32),
    mesh=_mesh,
    compiler_params=_sc_params,
    scratch_types=[
        pltpu.VMEM((EPT,), jnp.int32),
        pltpu.VMEM((EPT,), jnp.int32),
        pltpu.VMEM((EPT, 16), jnp.float32),
        pltpu.VMEM((128, 16), jnp.float32),
        pltpu.VMEM_SHARED((NPAD, 16), jnp.float32),
        pltpu.SemaphoreType.DMA,
        pltpu.SemaphoreType.DMA,
        pltpu.SemaphoreType.DMA,
    ],
)
def _agg_kernel(ei_hbm, feat_hbm, out_hbm,
                src_v, dst_v, rows_v, zer_v, acc_sh, semg, sems, semi):
    c = lax.axis_index("c")
    s = lax.axis_index("s")
    wid = c * 16 + s
    zero = jnp.zeros((16,), jnp.float32)

    pltpu.async_copy(ei_hbm.at[pl.ds(wid * EPT, EPT)], src_v, semi)
    pltpu.async_copy(ei_hbm.at[pl.ds(E + wid * EPT, EPT)], dst_v, semi)

    def fill_zero(i, _):
        zer_v[i] = zero
        return 0

    lax.fori_loop(0, 128, fill_zero, 0)

    def zero_acc(t, _):
        pltpu.sync_copy(zer_v, acc_sh.at[pl.ds(s * RPT + t * 128, 128)])
        return 0

    lax.fori_loop(0, RPT // 128, zero_acc, 0)
    pltpu.make_async_copy(ei_hbm.at[pl.ds(wid * EPT, EPT)], src_v, semi).wait()
    pltpu.make_async_copy(ei_hbm.at[pl.ds(wid * EPT, EPT)], dst_v, semi).wait()
    plsc.subcore_barrier()

    PH = [0, 10, 20, 30, K]

    def fire_g(j, _):
        pltpu.async_copy(feat_hbm.at[src_v.at[pl.ds(j * CH, CH)]],
                         rows_v.at[pl.ds(j * CH, CH)], semg)
        return 0

    def fire_s(j, _):
        pltpu.async_copy(rows_v.at[pl.ds(j * CH, CH)],
                         acc_sh.at[dst_v.at[pl.ds(j * CH, CH)]], sems, add=True)
        return 0

    lax.fori_loop(PH[0], PH[1], fire_g, 0)
    for p in range(4):
        lo, hi = PH[p], PH[p + 1]
        nrows = (hi - lo) * CH + (TAIL if p == 3 else 0)
        if p == 3:
            pltpu.async_copy(feat_hbm.at[src_v.at[pl.ds(K * CH, TAIL)]],
                             rows_v.at[pl.ds(K * CH, TAIL)], semg)
        pltpu.make_async_copy(feat_hbm.at[pl.ds(0, nrows)],
                              rows_v.at[pl.ds(lo * CH, nrows)], semg).wait()
        if p < 3:
            lax.fori_loop(PH[p + 1], PH[p + 2], fire_g, 0)
        lax.fori_loop(lo, hi, fire_s, 0)
        if p == 3:
            pltpu.async_copy(rows_v.at[pl.ds(K * CH, TAIL)],
                             acc_sh.at[dst_v.at[pl.ds(K * CH, TAIL)]],
                             sems, add=True)
    pltpu.make_async_copy(rows_v, acc_sh.at[pl.ds(0, EPT)], sems).wait()
    plsc.subcore_barrier()
    sl = pl.ds(s * RPT, RPT)
    pltpu.sync_copy(acc_sh.at[sl], out_hbm.at[c, sl])


def _tc1a_body(x_ref, w1_ref, xw_ref):
    xw_ref[0:N, :] = jnp.dot(x_ref[...], w1_ref[...],
                             preferred_element_type=jnp.float32)
    xw_ref[N:NPAD, :] = jnp.zeros((NPAD - N, 16), jnp.float32)


def _tc1b_body(xw_ref, dinvw_ref, xs_ref):
    xs_ref[...] = xw_ref[...] * dinvw_ref[...]


def _tc2_body(accp_ref, xs_ref, dinvw_ref, b1_ref, hs_ref):
    a = accp_ref[0] + accp_ref[1]
    dw = dinvw_ref[...]
    b1w = jnp.concatenate([b1_ref[...]] * 8)
    h = jnp.maximum(dw * (a + xs_ref[...]) + b1w, 0.0)
    hs_ref[...] = h * dw


def _tc3_body(accp2_ref, hs_ref, dinvw_ref, w2big_ref, b2_ref, out_ref):
    m = dinvw_ref[...] * (accp2_ref[0] + accp2_ref[1] + hs_ref[...])
    b2w = jnp.concatenate([b2_ref[...]] * 8)
    out_ref[...] = jnp.dot(m, w2big_ref[...],
                           preferred_element_type=jnp.float32) + b2w


def kernel(x, edge_index, W1, b1, W2, b2):
    f32 = jnp.float32
    ei_lin = edge_index.astype(jnp.int32).reshape(2 * E)

    dinvw = _deg_kernel(ei_lin)

    dinvw_w = dinvw.reshape(NW8, 128)
    xw = pl.pallas_call(
        _tc1a_body,
        out_shape=jax.ShapeDtypeStruct((NPAD, 16), f32),
    )(x, W1)
    xs_w = pl.pallas_call(
        _tc1b_body,
        out_shape=jax.ShapeDtypeStruct((NW8, 128), f32),
    )(xw.reshape(NW8, 128), dinvw_w)
    xs = xs_w.reshape(NPAD, 16)

    accp = _agg_kernel(ei_lin, xs)

    hs_w = pl.pallas_call(
        _tc2_body,
        out_shape=jax.ShapeDtypeStruct((NW8, 128), f32),
    )(accp.reshape(2, NW8, 128), xs_w, dinvw_w, b1)

    accp2 = _agg_kernel(ei_lin, hs_w.reshape(NPAD, 16))

    w2big = jnp.kron(jnp.eye(8, dtype=f32), W2.astype(f32))
    out_w = pl.pallas_call(
        _tc3_body,
        out_shape=jax.ShapeDtypeStruct((NW8, 16), f32),
    )(accp2.reshape(2, NW8, 128), hs_w, dinvw_w, w2big, b2)
    return out_w[:N * 2 // 16].reshape(N, 2)


# speedup vs baseline: 1.0039x; 1.0039x over previous
import functools

import jax
import jax.numpy as jnp
from jax import lax
from jax.experimental import pallas as pl
from jax.experimental.pallas import tpu as pltpu
from jax.experimental.pallas import tpu_sc as plsc

N = 10000
NPAD = 10240
NW8 = NPAD // 8
E = 160000
NW = 32
EPT = E // NW
CH = 128
K = EPT // CH
TAIL = EPT - K * CH
EPT_D = E // 16
K_D = EPT_D // CH
TAIL_D = EPT_D - K_D * CH
RPT = NPAD // 16
RPW = NPAD // 32

_mesh = plsc.VectorSubcoreMesh(core_axis_name="c", subcore_axis_name="s")
_sc_params = pltpu.CompilerParams(use_tc_tiling_on_sc=False,
                                  needs_layout_passes=False)


@functools.partial(
    pl.kernel,
    out_type=jax.ShapeDtypeStruct((NPAD, 16), jnp.float32),
    mesh=_mesh,
    compiler_params=_sc_params,
    scratch_types=[
        pltpu.VMEM((EPT_D,), jnp.int32),
        pltpu.VMEM((128,), jnp.float32),
        pltpu.VMEM((RPT,), jnp.float32),
        pltpu.VMEM((RPW,), jnp.float32),
        pltpu.VMEM((RPW, 16), jnp.float32),
        pltpu.VMEM_SHARED((NPAD,), jnp.float32),
        pltpu.SemaphoreType.DMA,
    ],
)
def _deg_kernel(ei_hbm, dinvw_hbm,
                idx_v, ones_v, zer_v, dinv_v, dvw_v, deg_sh, sem):
    c = lax.axis_index("c")
    s = lax.axis_index("s")
    wid = c * 16 + s
    one = jnp.ones((16,), jnp.float32)
    zero = jnp.zeros((16,), jnp.float32)

    def fill_ones(i, _):
        ones_v[pl.ds(i * 16, 16)] = one
        return 0

    lax.fori_loop(0, 8, fill_ones, 0)

    def fill_zero(i, _):
        zer_v[pl.ds(i * 16, 16)] = zero
        return 0

    lax.fori_loop(0, RPT // 16, fill_zero, 0)
    pltpu.sync_copy(ei_hbm.at[pl.ds(E + s * EPT_D, EPT_D)], idx_v)
    pltpu.sync_copy(zer_v, deg_sh.at[pl.ds(s * RPT, RPT)])
    plsc.subcore_barrier()

    def fire(j, _):
        pltpu.async_copy(ones_v.at[pl.ds(0, CH)],
                         deg_sh.at[idx_v.at[pl.ds(j * CH, CH)]], sem, add=True)
        return 0

    lax.fori_loop(0, K_D, fire, 0)
    pltpu.async_copy(ones_v.at[pl.ds(0, TAIL_D)],
                     deg_sh.at[idx_v.at[pl.ds(K_D * CH, TAIL_D)]], sem,
                     add=True)

    def drain(j, _):
        pltpu.make_async_copy(ones_v.at[pl.ds(0, CH)],
                              deg_sh.at[idx_v.at[pl.ds(j * CH, CH)]],
                              sem).wait()
        return 0

    lax.fori_loop(0, K_D, drain, 0)
    pltpu.make_async_copy(ones_v.at[pl.ds(0, TAIL_D)],
                          deg_sh.at[idx_v.at[pl.ds(K_D * CH, TAIL_D)]],
                          sem).wait()
    plsc.subcore_barrier()

    pltpu.sync_copy(deg_sh.at[pl.ds(wid * RPW, RPW)], dinv_v)

    def rsqrt_chunk(i, _):
        d = dinv_v[pl.ds(i * 16, 16)] + 1.0
        bits = plsc.bitcast(d, jnp.int32)
        y = plsc.bitcast(0x5F3759DF - lax.shift_right_logical(bits, 1),
                         jnp.float32)
        half = -0.5 * d
        y = y * (1.5 + half * y * y)
        y = y * (1.5 + half * y * y)
        y = y * (1.5 + half * y * y)
        dinv_v[pl.ds(i * 16, 16)] = y
        return 0

    lax.fori_loop(0, RPW // 16, rsqrt_chunk, 0)

    def splat_row(r, _):
        dvw_v[r] = plsc.load_gather(dinv_v, [jnp.full((16,), r, jnp.int32)])
        return 0

    lax.fori_loop(0, RPW, splat_row, 0)
    pltpu.sync_copy(dvw_v, dinvw_hbm.at[pl.ds(wid * RPW, RPW)])


@functools.partial(
    pl.kernel,
    out_type=jax.ShapeDtypeStruct((2, NPAD, 16), jnp.float32),
    mesh=_mesh,
    compiler_params=_sc_params,
    scratch_types=[
        pltpu.VMEM((EPT,), jnp.int32),
        pltpu.VMEM((EPT,), jnp.int32),
        pltpu.VMEM((EPT, 16), jnp.float32),
        pltpu.VMEM((128, 16), jnp.float32),
        pltpu.VMEM_SHARED((NPAD, 16), jnp.float32),
        pltpu.SemaphoreType.DMA,
        pltpu.SemaphoreType.DMA,
        pltpu.SemaphoreType.DMA,
    ],
)
def _agg_kernel(ei_hbm, feat_hbm, out_hbm,
                src_v, dst_v, rows_v, zer_v, acc_sh, semg, sems, semi):
    c = lax.axis_index("c")
    s = lax.axis_index("s")
    wid = c * 16 + s
    zero = jnp.zeros((16,), jnp.float32)

    pltpu.async_copy(ei_hbm.at[pl.ds(wid * EPT, EPT)], src_v, semi)
    pltpu.async_copy(ei_hbm.at[pl.ds(E + wid * EPT, EPT)], dst_v, semi)

    def fill_zero(i, _):
        zer_v[i] = zero
        return 0

    lax.fori_loop(0, 128, fill_zero, 0)

    def zero_acc(t, _):
        pltpu.sync_copy(zer_v, acc_sh.at[pl.ds(s * RPT + t * 128, 128)])
        return 0

    lax.fori_loop(0, RPT // 128, zero_acc, 0)
    pltpu.make_async_copy(ei_hbm.at[pl.ds(wid * EPT, EPT)], src_v, semi).wait()
    pltpu.make_async_copy(ei_hbm.at[pl.ds(wid * EPT, EPT)], dst_v, semi).wait()
    plsc.subcore_barrier()

    KH = K // 2

    def fire_g(j, _):
        pltpu.async_copy(feat_hbm.at[src_v.at[pl.ds(j * CH, CH)]],
                         rows_v.at[pl.ds(j * CH, CH)], semg)
        return 0

    def fire_s(j, _):
        pltpu.async_copy(rows_v.at[pl.ds(j * CH, CH)],
                         acc_sh.at[dst_v.at[pl.ds(j * CH, CH)]], sems, add=True)
        return 0

    lax.fori_loop(0, KH, fire_g, 0)
    pltpu.make_async_copy(feat_hbm.at[pl.ds(0, KH * CH)],
                          rows_v.at[pl.ds(0, KH * CH)], semg).wait()
    lax.fori_loop(0, KH, fire_s, 0)
    lax.fori_loop(KH, K, fire_g, 0)
    pltpu.async_copy(feat_hbm.at[src_v.at[pl.ds(K * CH, TAIL)]],
                     rows_v.at[pl.ds(K * CH, TAIL)], semg)
    pltpu.make_async_copy(feat_hbm.at[pl.ds(0, EPT - KH * CH)],
                          rows_v.at[pl.ds(KH * CH, EPT - KH * CH)], semg).wait()
    lax.fori_loop(KH, K, fire_s, 0)
    pltpu.async_copy(rows_v.at[pl.ds(K * CH, TAIL)],
                     acc_sh.at[dst_v.at[pl.ds(K * CH, TAIL)]], sems, add=True)
    pltpu.make_async_copy(rows_v, acc_sh.at[pl.ds(0, EPT)], sems).wait()
    plsc.subcore_barrier()
    sl = pl.ds(s * RPT, RPT)
    pltpu.sync_copy(acc_sh.at[sl], out_hbm.at[c, sl])


def _tc1a_body(x_ref, w1_ref, xw_ref):
    xw_ref[0:N, :] = jnp.dot(x_ref[...], w1_ref[...],
                             preferred_element_type=jnp.float32)
    xw_ref[N:NPAD, :] = jnp.zeros((NPAD - N, 16), jnp.float32)


def _tc1b_body(xw_ref, dinvw_ref, xs_ref):
    xs_ref[...] = xw_ref[...] * dinvw_ref[...]


def _tc2_body(accp_ref, xs_ref, dinvw_ref, b1_ref, hs_ref):
    a = accp_ref[0] + accp_ref[1]
    dw = dinvw_ref[...]
    b1w = jnp.concatenate([b1_ref[...]] * 8)
    h = jnp.maximum(dw * (a + xs_ref[...]) + b1w, 0.0)
    hs_ref[...] = h * dw


def _tc3_body(accp2_ref, hs_ref, dinvw_ref, w2big_ref, b2_ref, out_ref):
    m = dinvw_ref[...] * (accp2_ref[0] + accp2_ref[1] + hs_ref[...])
    b2w = jnp.concatenate([b2_ref[...]] * 8)
    out_ref[...] = jnp.dot(m, w2big_ref[...],
                           preferred_element_type=jnp.float32) + b2w


def kernel(x, edge_index, W1, b1, W2, b2):
    f32 = jnp.float32
    ei_lin = edge_index.astype(jnp.int32).reshape(2 * E)

    dinvw = _deg_kernel(ei_lin)

    dinvw_w = dinvw.reshape(NW8, 128)
    xw = pl.pallas_call(
        _tc1a_body,
        out_shape=jax.ShapeDtypeStruct((NPAD, 16), f32),
    )(x, W1)
    xs_w = pl.pallas_call(
        _tc1b_body,
        out_shape=jax.ShapeDtypeStruct((NW8, 128), f32),
    )(xw.reshape(NW8, 128), dinvw_w)
    xs = xs_w.reshape(NPAD, 16)

    accp = _agg_kernel(ei_lin, xs)

    hs_w = pl.pallas_call(
        _tc2_body,
        out_shape=jax.ShapeDtypeStruct((NW8, 128), f32),
    )(accp.reshape(2, NW8, 128), xs_w, dinvw_w, b1)

    accp2 = _agg_kernel(ei_lin, hs_w.reshape(NPAD, 16))

    w2big = jnp.kron(jnp.eye(8, dtype=f32), W2.astype(f32))
    out_w = pl.pallas_call(
        _tc3_body,
        out_shape=jax.ShapeDtypeStruct((NW8, 16), f32),
    )(accp2.reshape(2, NW8, 128), hs_w, dinvw_w, w2big, b2)
    return out_w[:N * 2 // 16].reshape(N, 2)
